# baseline (speedup 1.0000x reference)
import jax
import jax.numpy as jnp
from jax import lax
from jax.experimental import pallas as pl
from jax.experimental.pallas import tpu as pltpu


_DOT_TRANS_B = (((1,), (1,)), ((), ()))

_TN = 256
_NH = 2


def _noisy_body(x_hbm, eps_oc_ref, eps_in_ref, mu_w_ref, sig_w_ref,
                mu_b_ref, sig_b_ref, eps_or_ref, o_ref,
                x_vmem, w_eff_ref, sem_x):
    B = x_vmem.shape[0]
    bh = B // _NH
    j = pl.program_id(0)
    h = pl.program_id(1)

    @pl.when(jnp.logical_and(j == 0, h == 0))
    def _start_x():
        for t in range(_NH):
            rs = pl.ds(t * bh, bh)
            pltpu.make_async_copy(x_hbm.at[rs, :], x_vmem.at[rs, :],
                                  sem_x.at[t]).start()
        rs0 = pl.ds(0, bh)
        pltpu.make_async_copy(x_vmem.at[rs0, :], x_vmem.at[rs0, :],
                              sem_x.at[0]).wait()

    @pl.when(jnp.logical_and(j == 0, h == 1))
    def _wait_x1():
        rs1 = pl.ds(bh, bh)
        pltpu.make_async_copy(x_vmem.at[rs1, :], x_vmem.at[rs1, :],
                              sem_x.at[1]).wait()

    @pl.when(h == 0)
    def _combine():
        scale = eps_oc_ref[...] * eps_in_ref[...]
        w_eff_ref[...] = mu_w_ref[...] + sig_w_ref[...] * scale

    rs = pl.ds(h * bh, bh)
    y = lax.dot_general(x_vmem[rs, :], w_eff_ref[...], _DOT_TRANS_B,
                        preferred_element_type=jnp.float32)
    b_eff = sig_b_ref[...] * eps_or_ref[...] + mu_b_ref[...]
    o_ref[...] = y + b_eff


def kernel(x, mu_weight, sigma_weight, mu_bias, sigma_bias, eps_in, eps_out):
    B, F_in = x.shape
    F_out = mu_bias.shape[0]

    x_f = x.astype(jnp.float32)
    mu_w = mu_weight.astype(jnp.float32)
    sig_w = sigma_weight.astype(jnp.float32)
    eps_in_row = eps_in.reshape(1, F_in).astype(jnp.float32)
    eps_out_col = eps_out.reshape(F_out, 1).astype(jnp.float32)
    eps_out_row = eps_out.reshape(1, F_out).astype(jnp.float32)
    mu_b_row = mu_bias.reshape(1, F_out).astype(jnp.float32)
    sig_b_row = sigma_bias.reshape(1, F_out).astype(jnp.float32)

    grid = (F_out // _TN, _NH)

    return pl.pallas_call(
        _noisy_body,
        out_shape=jax.ShapeDtypeStruct((B, F_out), jnp.float32),
        grid=grid,
        in_specs=[
            pl.BlockSpec(memory_space=pl.ANY),
            pl.BlockSpec((_TN, 1), lambda j, h: (j, 0)),
            pl.BlockSpec((1, F_in), lambda j, h: (0, 0)),
            pl.BlockSpec((_TN, F_in), lambda j, h: (j, 0)),
            pl.BlockSpec((_TN, F_in), lambda j, h: (j, 0)),
            pl.BlockSpec((1, _TN), lambda j, h: (0, j)),
            pl.BlockSpec((1, _TN), lambda j, h: (0, j)),
            pl.BlockSpec((1, _TN), lambda j, h: (0, j)),
        ],
        out_specs=pl.BlockSpec((B // _NH, _TN), lambda j, h: (h, j)),
        scratch_shapes=[
            pltpu.VMEM((B, F_in), jnp.float32),
            pltpu.VMEM((_TN, F_in), jnp.float32),
            pltpu.SemaphoreType.DMA((_NH,)),
        ],
        compiler_params=pltpu.CompilerParams(
            dimension_semantics=("arbitrary", "arbitrary"),
            vmem_limit_bytes=64 * 1024 * 1024,
        ),
    )(x_f, eps_out_col, eps_in_row, mu_w, sig_w, mu_b_row, sig_b_row,
      eps_out_row)


# speedup vs baseline: 1.2835x; 1.2835x over previous
import jax
import jax.numpy as jnp
from jax import lax
from jax.experimental import pallas as pl
from jax.experimental.pallas import tpu as pltpu


_DOT_TRANS_B = (((1,), (1,)), ((), ()))

_TN = 256


def _noisy_body(x_ref, mu_w_ref, sig_w_ref, eps_oc_ref, eps_in_ref,
                mu_b_ref, sig_b_ref, eps_or_ref, o_ref, w_eff0, w_eff1):
    nt = pl.num_programs(0) - 1
    j = pl.program_id(0)
    even = lax.rem(j, 2) == 0

    def combine():
        scale = eps_oc_ref[...] * eps_in_ref[...]
        return mu_w_ref[...] + sig_w_ref[...] * scale

    @pl.when(jnp.logical_and(j < nt, even))
    def _combine_even():
        w_eff0[...] = combine()

    @pl.when(jnp.logical_and(j < nt, jnp.logical_not(even)))
    def _combine_odd():
        w_eff1[...] = combine()

    def emit(w_eff_ref):
        y = lax.dot_general(x_ref[...], w_eff_ref[...], _DOT_TRANS_B,
                            preferred_element_type=jnp.float32)
        b_eff = sig_b_ref[...] * eps_or_ref[...] + mu_b_ref[...]
        o_ref[...] = y + b_eff

    @pl.when(jnp.logical_and(j > 0, jnp.logical_not(even)))
    def _dot_from_even():
        emit(w_eff0)

    @pl.when(jnp.logical_and(j > 0, even))
    def _dot_from_odd():
        emit(w_eff1)


def kernel(x, mu_weight, sigma_weight, mu_bias, sigma_bias, eps_in, eps_out):
    B, F_in = x.shape
    F_out = mu_bias.shape[0]
    nt = F_out // _TN

    x_f = x.astype(jnp.float32)
    mu_w = mu_weight.astype(jnp.float32)
    sig_w = sigma_weight.astype(jnp.float32)
    eps_in_row = eps_in.reshape(1, F_in).astype(jnp.float32)
    eps_out_col = eps_out.reshape(F_out, 1).astype(jnp.float32)
    eps_out_row = eps_out.reshape(1, F_out).astype(jnp.float32)
    mu_b_row = mu_bias.reshape(1, F_out).astype(jnp.float32)
    sig_b_row = sigma_bias.reshape(1, F_out).astype(jnp.float32)

    last = nt - 1
    wmap = lambda j: (min(j, last) if isinstance(j, int) else jnp.minimum(j, last), 0)
    omap = lambda j: (0, max(j - 1, 0) if isinstance(j, int) else jnp.maximum(j - 1, 0))

    return pl.pallas_call(
        _noisy_body,
        out_shape=jax.ShapeDtypeStruct((B, F_out), jnp.float32),
        grid=(nt + 1,),
        in_specs=[
            pl.BlockSpec((B, F_in), lambda j: (0, 0)),
            pl.BlockSpec((_TN, F_in), wmap),
            pl.BlockSpec((_TN, F_in), wmap),
            pl.BlockSpec((_TN, 1), wmap),
            pl.BlockSpec((1, F_in), lambda j: (0, 0)),
            pl.BlockSpec((1, _TN), omap),
            pl.BlockSpec((1, _TN), omap),
            pl.BlockSpec((1, _TN), omap),
        ],
        out_specs=pl.BlockSpec((B, _TN), omap),
        scratch_shapes=[
            pltpu.VMEM((_TN, F_in), jnp.float32),
            pltpu.VMEM((_TN, F_in), jnp.float32),
        ],
        compiler_params=pltpu.CompilerParams(
            dimension_semantics=("arbitrary",),
            vmem_limit_bytes=64 * 1024 * 1024,
        ),
    )(x_f, mu_w, sig_w, eps_out_col, eps_in_row, mu_b_row, sig_b_row,
      eps_out_row)
